# baseline (speedup 1.0000x reference)
import jax
import jax.numpy as jnp
from jax.experimental import pallas as pl
from jax.experimental.pallas import tpu as pltpu

B, D, H, K, L = 1024, 32, 64, 512, 3
K_TILE = 64
NK = K // K_TILE


def _enc_kernel(x_ref, bw_ref, bb_ref, w_ref, lb_ref,
                enc_ref, out_ref,
                cur_ref, u_ref, min_ref, idx_ref, delta_ref):
    i = pl.program_id(0)
    k = pl.program_id(1)

    @pl.when(jnp.logical_and(i == 0, k == 0))
    def _init():
        cur_ref[...] = jnp.zeros_like(cur_ref)

    @pl.when(k == 0)
    def _stage_start():
        cur = cur_ref[...]
        u = jnp.dot(cur, bw_ref[...], preferred_element_type=jnp.float32)
        u = jnp.maximum(u + bb_ref[...], 0.0)
        u_ref[...] = u
        min_ref[...] = jnp.full_like(min_ref, jnp.inf)
        idx_ref[...] = jnp.zeros_like(idx_ref)
        delta_ref[...] = jnp.zeros_like(delta_ref)

    u = u_ref[...]
    mm = jnp.dot(u, w_ref[0], preferred_element_type=jnp.float32)
    delta = lb_ref[0][None, :, :] + mm.reshape(B, K_TILE, D)
    cur = cur_ref[...]
    cand = cur[:, None, :] + delta
    diff = cand - x_ref[...][:, None, :]
    losses = jnp.mean(diff * diff, axis=-1)

    tmin = jnp.min(losses, axis=-1, keepdims=True)
    targ = jnp.argmin(losses, axis=-1).astype(jnp.int32)[:, None]
    onehot3 = (jax.lax.broadcasted_iota(jnp.int32, (B, K_TILE, D), 1)
               == targ[:, :, None])
    tdelta = jnp.sum(jnp.where(onehot3, delta, 0.0), axis=1)

    upd = tmin < min_ref[...]
    min_ref[...] = jnp.where(upd, tmin, min_ref[...])
    idx_ref[...] = jnp.where(upd, targ + k * K_TILE, idx_ref[...])
    delta_ref[...] = jnp.where(upd, tdelta, delta_ref[...])

    @pl.when(k == NK - 1)
    def _stage_end():
        enc_ref[0] = idx_ref[...]
        newcur = cur_ref[...] + delta_ref[...]
        cur_ref[...] = newcur

        @pl.when(i == L - 1)
        def _done():
            out_ref[...] = newcur


def kernel(inputs, base_W, base_b, layer_Ws, layer_biases):
    enc, cur = pl.pallas_call(
        _enc_kernel,
        grid=(L, NK),
        in_specs=[
            pl.BlockSpec((B, D), lambda i, k: (0, 0)),
            pl.BlockSpec((D, H), lambda i, k: (0, 0)),
            pl.BlockSpec((1, H), lambda i, k: (0, 0)),
            pl.BlockSpec((1, H, K_TILE * D), lambda i, k: (i, 0, k)),
            pl.BlockSpec((1, K_TILE, D), lambda i, k: (i, k, 0)),
        ],
        out_specs=[
            pl.BlockSpec((1, B, 1), lambda i, k: (i, 0, 0)),
            pl.BlockSpec((B, D), lambda i, k: (0, 0)),
        ],
        out_shape=[
            jax.ShapeDtypeStruct((L, B, 1), jnp.int32),
            jax.ShapeDtypeStruct((B, D), jnp.float32),
        ],
        scratch_shapes=[
            pltpu.VMEM((B, D), jnp.float32),
            pltpu.VMEM((B, H), jnp.float32),
            pltpu.VMEM((B, 1), jnp.float32),
            pltpu.VMEM((B, 1), jnp.int32),
            pltpu.VMEM((B, D), jnp.float32),
        ],
    )(inputs, base_W, base_b.reshape(1, H), layer_Ws, layer_biases)
    return enc[:, :, 0].T, cur
